# Initial kernel scaffold; baseline (speedup 1.0000x reference)
import jax
import jax.numpy as jnp
from jax.experimental import pallas as pl


def kernel(x, edge_index, W1, b1, W2, b2, Wa, ba, Wr, br):
    raise NotImplementedError("write your pallas kernel here")


# speedup vs baseline: 21.7031x; 21.7031x over previous
import functools

import jax
import jax.numpy as jnp
from jax import lax
from jax.experimental import pallas as pl
from jax.experimental.pallas import tpu as pltpu
from jax.experimental.pallas import tpu_sc as plsc

N = 10000
D = 128
E = 320000
HID = 128

NC = 2
NS = 16
NW = NC * NS
EPW = E // NW
C = 125
K = EPW // C
RPT = N // NS

_mesh = plsc.VectorSubcoreMesh(core_axis_name="c", subcore_axis_name="s")


@functools.partial(
    pl.kernel,
    out_type=jax.ShapeDtypeStruct((NC, NS, RPT, 16), jnp.float32),
    mesh=_mesh,
    scratch_types=[
        pltpu.VMEM((K, C), jnp.int32),
        pltpu.VMEM((C, 16), jnp.float32),
        pltpu.VMEM_SHARED((N, 16), jnp.float32),
    ],
)
def _sc_degree(dst_hbm, out_hbm, idx_v, ones_v, acc_sh):
    cid = lax.axis_index("c")
    sid = lax.axis_index("s")
    wid = sid * NC + cid
    pltpu.sync_copy(dst_hbm.at[wid], idx_v)

    z16 = jnp.zeros((16,), jnp.float32)

    def zfill(i, carry):
        ones_v[i, :] = z16
        return carry

    lax.fori_loop(0, C, zfill, 0)
    for r in range(RPT // C):
        pltpu.sync_copy(ones_v, acc_sh.at[pl.ds(sid * RPT + r * C, C)])

    o16 = jnp.ones((16,), jnp.float32)

    def ofill(i, carry):
        ones_v[i, :] = o16
        return carry

    lax.fori_loop(0, C, ofill, 0)
    plsc.subcore_barrier()

    def body(j, carry):
        pltpu.sync_copy(ones_v, acc_sh.at[idx_v.at[j]], add=True)
        return carry

    lax.fori_loop(0, K, body, 0)
    plsc.subcore_barrier()

    pltpu.sync_copy(acc_sh.at[pl.ds(sid * RPT, RPT)], out_hbm.at[cid, sid])


@functools.partial(
    pl.kernel,
    out_type=jax.ShapeDtypeStruct((NC, NS, RPT, HID), jnp.float32),
    mesh=_mesh,
    scratch_types=[
        pltpu.VMEM((K, C), jnp.int32),
        pltpu.VMEM((K, C), jnp.int32),
        pltpu.VMEM((C, HID), jnp.float32),
        pltpu.VMEM_SHARED((N, HID), jnp.float32),
        pltpu.SemaphoreType.DMA,
    ],
)
def _sc_segsum(u_hbm, src_hbm, dst_hbm, out_hbm, src_v, dst_v, stage_v,
               acc_sh, sem):
    cid = lax.axis_index("c")
    sid = lax.axis_index("s")
    wid = sid * NC + cid
    pltpu.sync_copy(src_hbm.at[wid], src_v)
    pltpu.sync_copy(dst_hbm.at[wid], dst_v)

    z16 = jnp.zeros((16,), jnp.float32)

    def zfill(i, carry):
        for t in range(HID // 16):
            stage_v[i, pl.ds(t * 16, 16)] = z16
        return carry

    lax.fori_loop(0, C, zfill, 0)
    for r in range(RPT // C):
        pltpu.sync_copy(stage_v, acc_sh.at[pl.ds(sid * RPT + r * C, C)])
    plsc.subcore_barrier()

    def body(j, carry):
        pltpu.async_copy(u_hbm.at[src_v.at[j]], stage_v, sem).wait()
        pltpu.sync_copy(stage_v, acc_sh.at[dst_v.at[j]], add=True)
        return carry

    lax.fori_loop(0, K, body, 0)
    plsc.subcore_barrier()

    pltpu.sync_copy(acc_sh.at[pl.ds(sid * RPT, RPT)], out_hbm.at[cid, sid])


def _k1_body(x_ref, w_ref, degp_ref, u_ref, dinv_ref):
    deg = degp_ref[0, :, 0] + degp_ref[1, :, 0] + 1.0
    dinv = lax.rsqrt(deg)
    xw = jnp.dot(x_ref[...], w_ref[...], preferred_element_type=jnp.float32)
    u_ref[...] = xw * dinv[:, None]
    dinv_ref[...] = dinv[:, None]


def _k2_body(s_ref, u_ref, dinv_ref, b_ref, w_ref, o_ref):
    s = s_ref[0] + s_ref[1] + u_ref[...]
    h = jnp.maximum(s * dinv_ref[...] + b_ref[...], 0.0)
    o_ref[...] = jnp.dot(h, w_ref[...],
                         preferred_element_type=jnp.float32) * dinv_ref[...]


def _k3_body(s_ref, u_ref, dinv_ref, b_ref, wc_ref, bc_ref, o_ref):
    h2 = (s_ref[0] + s_ref[1] + u_ref[...]) * dinv_ref[...] + b_ref[...]
    o_ref[...] = jnp.dot(h2, wc_ref[...],
                         preferred_element_type=jnp.float32) + bc_ref[...]


_k1 = pl.pallas_call(
    _k1_body,
    out_shape=(jax.ShapeDtypeStruct((N, HID), jnp.float32),
               jax.ShapeDtypeStruct((N, 1), jnp.float32)),
)

_k2 = pl.pallas_call(
    _k2_body,
    out_shape=jax.ShapeDtypeStruct((N, HID), jnp.float32),
)

_k3 = pl.pallas_call(
    _k3_body,
    out_shape=jax.ShapeDtypeStruct((N, HID), jnp.float32),
)


def kernel(x, edge_index, W1, b1, W2, b2, Wa, ba, Wr, br):
    src3 = edge_index[0].reshape(NW, K, C)
    dst3 = edge_index[1].reshape(NW, K, C)

    degp = _sc_degree(dst3).reshape(NC, N, 16)
    u1, dinv = _k1(x, W1, degp)
    s1 = _sc_segsum(u1, src3, dst3).reshape(NC, N, HID)
    u2 = _k2(s1, u1, dinv, b1.reshape(1, HID), W2)
    s2 = _sc_segsum(u2, src3, dst3).reshape(NC, N, HID)
    Wc = jnp.concatenate([Wa, Wr], axis=1)
    bc = jnp.concatenate([ba, br]).reshape(1, HID)
    return _k3(s2, u2, dinv, b2.reshape(1, HID), Wc, bc)


# speedup vs baseline: 27.0030x; 1.2442x over previous
import functools

import jax
import jax.numpy as jnp
from jax import lax
from jax.experimental import pallas as pl
from jax.experimental.pallas import tpu as pltpu
from jax.experimental.pallas import tpu_sc as plsc

N = 10000
D = 128
E = 320000
HID = 128

NC = 2
NS = 16
NW = NC * NS
EPW = E // NW
C = 125
K = EPW // C
RPT = N // NS

_mesh = plsc.VectorSubcoreMesh(core_axis_name="c", subcore_axis_name="s")


@functools.partial(
    pl.kernel,
    out_type=jax.ShapeDtypeStruct((NC, NS, RPT, 16), jnp.float32),
    mesh=_mesh,
    scratch_types=[
        pltpu.VMEM((K, C), jnp.int32),
        pltpu.VMEM((C, 16), jnp.float32),
        pltpu.VMEM_SHARED((N, 16), jnp.float32),
    ],
)
def _sc_degree(dst_hbm, out_hbm, idx_v, ones_v, acc_sh):
    cid = lax.axis_index("c")
    sid = lax.axis_index("s")
    wid = sid * NC + cid
    pltpu.sync_copy(dst_hbm.at[wid], idx_v)

    z16 = jnp.zeros((16,), jnp.float32)

    def zfill(i, carry):
        ones_v[i, :] = z16
        return carry

    lax.fori_loop(0, C, zfill, 0)
    for r in range(RPT // C):
        pltpu.sync_copy(ones_v, acc_sh.at[pl.ds(sid * RPT + r * C, C)])

    o16 = jnp.ones((16,), jnp.float32)

    def ofill(i, carry):
        ones_v[i, :] = o16
        return carry

    lax.fori_loop(0, C, ofill, 0)
    plsc.subcore_barrier()

    def body(j, carry):
        pltpu.sync_copy(ones_v, acc_sh.at[idx_v.at[j]], add=True)
        return carry

    lax.fori_loop(0, K, body, 0)
    plsc.subcore_barrier()

    pltpu.sync_copy(acc_sh.at[pl.ds(sid * RPT, RPT)], out_hbm.at[cid, sid])


@functools.partial(
    pl.kernel,
    out_type=jax.ShapeDtypeStruct((NC, NS, RPT, HID), jnp.float32),
    mesh=_mesh,
    scratch_types=[
        pltpu.VMEM((K // 2, C), jnp.int32),
        pltpu.VMEM((K // 2, C), jnp.int32),
        pltpu.VMEM((C, HID), jnp.float32),
        pltpu.VMEM((C, HID), jnp.float32),
        pltpu.VMEM_SHARED((N, HID), jnp.float32),
    ],
)
def _sc_segsum(u_hbm, src_hbm, dst_hbm, out_hbm, src_v, dst_v, b0, b1,
               acc_sh):
    cid = lax.axis_index("c")
    sid = lax.axis_index("s")
    wid = sid * NC + cid
    KH = K // 2

    z16 = jnp.zeros((16,), jnp.float32)

    def zfill(i, carry):
        for t in range(HID // 16):
            b0[i, pl.ds(t * 16, 16)] = z16
        return carry

    lax.fori_loop(0, C, zfill, 0)
    for r in range(RPT // C):
        pltpu.sync_copy(b0, acc_sh.at[pl.ds(sid * RPT + r * C, C)])
    plsc.subcore_barrier()

    def gstart(j, buf, sem):
        return pltpu.async_copy(u_hbm.at[src_v.at[j]], buf, sem)

    def gwait(j, buf, sem):
        pltpu.make_async_copy(u_hbm.at[src_v.at[j]], buf, sem).wait()

    def sstart(j, buf, sem):
        return pltpu.async_copy(buf, acc_sh.at[dst_v.at[j]], sem, add=True)

    def swait(j, buf, sem):
        pltpu.make_async_copy(buf, acc_sh.at[dst_v.at[j]], sem).wait()

    def _run(g0, g1, s0, s1):
        for h in range(2):
            pltpu.sync_copy(src_hbm.at[wid, pl.ds(h * KH, KH)], src_v)
            pltpu.sync_copy(dst_hbm.at[wid, pl.ds(h * KH, KH)], dst_v)

            gstart(0, b0, g0).wait()
            sstart(0, b0, s0)
            gstart(1, b1, g1)

            def body(jj, carry):
                j = 2 * jj
                gwait(j + 1, b1, g1)
                sb = sstart(j + 1, b1, s1)
                swait(j, b0, s0)
                gstart(j + 2, b0, g0).wait()
                sstart(j + 2, b0, s0)
                sb.wait()
                gstart(j + 3, b1, g1)
                return carry

            lax.fori_loop(0, KH // 2 - 1, body, 0)
            gwait(KH - 1, b1, g1)
            sb = sstart(KH - 1, b1, s1)
            swait(KH - 2, b0, s0)
            sb.wait()

    pl.run_scoped(_run, g0=pltpu.SemaphoreType.DMA(()),
                  g1=pltpu.SemaphoreType.DMA(()),
                  s0=pltpu.SemaphoreType.DMA(()),
                  s1=pltpu.SemaphoreType.DMA(()))

    plsc.subcore_barrier()
    pltpu.sync_copy(acc_sh.at[pl.ds(sid * RPT, RPT)], out_hbm.at[cid, sid])


def _k1_body(x_ref, w_ref, degp_ref, u_ref, dinv_ref):
    deg = degp_ref[0, :, 0] + degp_ref[1, :, 0] + 1.0
    dinv = lax.rsqrt(deg)
    xw = jnp.dot(x_ref[...], w_ref[...], preferred_element_type=jnp.float32)
    u_ref[...] = xw * dinv[:, None]
    dinv_ref[...] = dinv[:, None]


def _k2_body(s_ref, u_ref, dinv_ref, b_ref, w_ref, o_ref):
    s = s_ref[0] + s_ref[1] + u_ref[...]
    h = jnp.maximum(s * dinv_ref[...] + b_ref[...], 0.0)
    o_ref[...] = jnp.dot(h, w_ref[...],
                         preferred_element_type=jnp.float32) * dinv_ref[...]


def _k3_body(s_ref, u_ref, dinv_ref, b_ref, wc_ref, bc_ref, o_ref):
    h2 = (s_ref[0] + s_ref[1] + u_ref[...]) * dinv_ref[...] + b_ref[...]
    o_ref[...] = jnp.dot(h2, wc_ref[...],
                         preferred_element_type=jnp.float32) + bc_ref[...]


_k1 = pl.pallas_call(
    _k1_body,
    out_shape=(jax.ShapeDtypeStruct((N, HID), jnp.float32),
               jax.ShapeDtypeStruct((N, 1), jnp.float32)),
)

_k2 = pl.pallas_call(
    _k2_body,
    out_shape=jax.ShapeDtypeStruct((N, HID), jnp.float32),
)

_k3 = pl.pallas_call(
    _k3_body,
    out_shape=jax.ShapeDtypeStruct((N, HID), jnp.float32),
)


def kernel(x, edge_index, W1, b1, W2, b2, Wa, ba, Wr, br):
    src3 = edge_index[0].reshape(NW, K, C)
    dst3 = edge_index[1].reshape(NW, K, C)

    degp = _sc_degree(dst3).reshape(NC, N, 16)
    u1, dinv = _k1(x, W1, degp)
    s1 = _sc_segsum(u1, src3, dst3).reshape(NC, N, HID)
    u2 = _k2(s1, u1, dinv, b1.reshape(1, HID), W2)
    s2 = _sc_segsum(u2, src3, dst3).reshape(NC, N, HID)
    Wc = jnp.concatenate([Wa, Wr], axis=1)
    bc = jnp.concatenate([ba, br]).reshape(1, HID)
    return _k3(s2, u2, dinv, b2.reshape(1, HID), Wc, bc)
